# baseline (speedup 1.0000x reference)
import functools

import jax
import jax.numpy as jnp
from jax import lax
from jax.experimental import pallas as pl
from jax.experimental.pallas import tpu as pltpu
from jax.experimental.pallas import tpu_sc as plsc

B, T, H, W = 3, 300, 224, 224
NUM_SAMPLES = 32
NC, NS = 2, 16
NW = NC * NS
FRAMES_PER_W = (B * NUM_SAMPLES) // NW
HALF = H // 2
NBUF = 4
NCHUNKS = 2 * FRAMES_PER_W


def _chunk_coords(c):
    def coords(base):
        r = base + c // 2
        b = r // NUM_SAMPLES
        j = r % NUM_SAMPLES
        t = (j * (T - 1)) // (NUM_SAMPLES - 1)
        return b, j, t, (c % 2) * HALF

    return coords


def _sc_body(x_hbm, out_hbm, b0, b1, b2, b3, si0, si1, si2, si3, so0, so1,
             so2, so3):
    wid = lax.axis_index("s") * NC + lax.axis_index("c")
    base = wid * FRAMES_PER_W
    bufs = (b0, b1, b2, b3)
    sem_in = (si0, si1, si2, si3)
    sem_out = (so0, so1, so2, so3)

    def start_in(c):
        b, _, t, h0 = _chunk_coords(c)(base)
        return pltpu.async_copy(
            x_hbm.at[b, t, pl.ds(h0, HALF)], bufs[c % NBUF], sem_in[c % NBUF])

    def start_out(c):
        b, j, _, h0 = _chunk_coords(c)(base)
        return pltpu.async_copy(
            bufs[c % NBUF], out_hbm.at[b, j, pl.ds(h0, HALF)],
            sem_out[c % NBUF])

    ins = [start_in(c) for c in range(NBUF)]
    outs = [None] * NCHUNKS
    for c in range(NCHUNKS):
        ins[c % NBUF].wait()
        outs[c] = start_out(c)
        d = c - (NBUF - 2)
        if d >= 0 and d + NBUF < NCHUNKS:
            outs[d].wait()
            ins[d % NBUF] = start_in(d + NBUF)
    for c in range(NCHUNKS - NBUF, NCHUNKS):
        outs[c].wait()


@jax.jit
def kernel(x):
    mesh = plsc.VectorSubcoreMesh(core_axis_name="c", subcore_axis_name="s")
    run = functools.partial(
        pl.kernel,
        mesh=mesh,
        out_type=jax.ShapeDtypeStruct((B, NUM_SAMPLES, H, W), jnp.float32),
        scratch_types=(
            [pltpu.VMEM((HALF, W), jnp.float32)] * NBUF
            + [pltpu.SemaphoreType.DMA] * (2 * NBUF)
        ),
    )(_sc_body)
    return run(x)


# speedup vs baseline: 1.0138x; 1.0138x over previous
import functools

import jax
import jax.numpy as jnp
from jax import lax
from jax.experimental import pallas as pl
from jax.experimental.pallas import tpu as pltpu
from jax.experimental.pallas import tpu_sc as plsc

B, T, H, W = 3, 300, 224, 224
NUM_SAMPLES = 32
NC, NS = 2, 16
NW = NC * NS
ROWS_PER_W = (B * NUM_SAMPLES) // NW


def _src_frame(r):
    b = r // NUM_SAMPLES
    j = r % NUM_SAMPLES
    t = (j * (T - 1)) // (NUM_SAMPLES - 1)
    return b, j, t


def _sc_body(x_hbm, out_hbm, buf_a, buf_b, sai, sbi, sao, sbo):
    wid = lax.axis_index("s") * NC + lax.axis_index("c")
    base = wid * ROWS_PER_W

    b0, j0, t0 = _src_frame(base)
    b1, j1, t1 = _src_frame(base + 1)
    b2, j2, t2 = _src_frame(base + 2)

    in0 = pltpu.async_copy(x_hbm.at[b0, t0], buf_a, sai)
    in1 = pltpu.async_copy(x_hbm.at[b1, t1], buf_b, sbi)
    in0.wait()
    out0 = pltpu.async_copy(buf_a, out_hbm.at[b0, j0], sao)
    in1.wait()
    out1 = pltpu.async_copy(buf_b, out_hbm.at[b1, j1], sbo)
    out0.wait()
    in2 = pltpu.async_copy(x_hbm.at[b2, t2], buf_a, sai)
    in2.wait()
    out2 = pltpu.async_copy(buf_a, out_hbm.at[b2, j2], sao)
    out1.wait()
    out2.wait()


@jax.jit
def kernel(x):
    mesh = plsc.VectorSubcoreMesh(core_axis_name="c", subcore_axis_name="s")
    run = functools.partial(
        pl.kernel,
        mesh=mesh,
        out_type=jax.ShapeDtypeStruct((B, NUM_SAMPLES, H, W), jnp.float32),
        scratch_types=[
            pltpu.VMEM((H, W), jnp.float32),
            pltpu.VMEM((H, W), jnp.float32),
            pltpu.SemaphoreType.DMA,
            pltpu.SemaphoreType.DMA,
            pltpu.SemaphoreType.DMA,
            pltpu.SemaphoreType.DMA,
        ],
    )(_sc_body)
    return run(x)
